# baseline (speedup 1.0000x reference)
import functools

import jax
import jax.numpy as jnp
from jax import lax
from jax.experimental import pallas as pl
from jax.experimental.pallas import tpu as pltpu
from jax.experimental.pallas import tpu_sc as plsc

NUM_TOKENS = 8192
DIM = 768

_info = plsc.get_sparse_core_info()
_NC = _info.num_cores
_NS = _info.num_subcores
_NW = _NC * _NS
_ROWS_PER_W = NUM_TOKENS // _NW
_S = 32
_K = 4
_NCHUNK = _ROWS_PER_W // _S


@functools.partial(
    pl.kernel,
    out_type=jax.ShapeDtypeStruct((NUM_TOKENS, 1, DIM), jnp.float32),
    mesh=plsc.VectorSubcoreMesh(core_axis_name="c", subcore_axis_name="s"),
    scratch_types=(
        [pltpu.VMEM((_S, DIM), jnp.float32) for _ in range(_K)]
        + [pltpu.SemaphoreType.DMA] * (2 * _NCHUNK)
    ),
)
def _identity_rows_sc(table_hbm, out_hbm, *scratch):
    slots = scratch[:_K]
    sem_in = scratch[_K:_K + _NCHUNK]
    sem_out = scratch[_K + _NCHUNK:]
    sid = lax.axis_index("s")
    wid = sid * _NC + lax.axis_index("c")
    base = wid * _ROWS_PER_W

    ins = [None] * _NCHUNK
    outs = [None] * _NCHUNK

    def start_in(j):
        ins[j] = pltpu.async_copy(
            table_hbm.at[pl.ds(base + j * _S, _S)], slots[j % _K], sem_in[j])

    def start_out(j):
        outs[j] = pltpu.async_copy(
            slots[j % _K], out_hbm.at[pl.ds(base + j * _S, _S), 0], sem_out[j])

    for j in range(_K):
        start_in(j)
    for j in range(_NCHUNK):
        ins[j].wait()
        start_out(j)
        if j + _K < _NCHUNK:
            outs[j].wait()
            start_in(j + _K)
    for j in range(_NCHUNK - _K, _NCHUNK):
        outs[j].wait()


def kernel(table):
    return _identity_rows_sc(table)


# speedup vs baseline: 1.0260x; 1.0260x over previous
import functools

import jax
import jax.numpy as jnp
from jax import lax
from jax.experimental import pallas as pl
from jax.experimental.pallas import tpu as pltpu
from jax.experimental.pallas import tpu_sc as plsc

NUM_TOKENS = 8192
DIM = 768

_NC = 2
_ROWS_PER_C = NUM_TOKENS // _NC
_S = 512
_K = 4
_NCHUNK = _ROWS_PER_C // _S


@functools.partial(
    pl.kernel,
    out_type=jax.ShapeDtypeStruct((NUM_TOKENS, 1, DIM), jnp.float32),
    mesh=plsc.ScalarSubcoreMesh(axis_name="c", num_cores=_NC),
    scratch_types=(
        [pltpu.VMEM_SHARED((_S, 1, DIM), jnp.float32) for _ in range(_K)]
        + [pltpu.SemaphoreType.DMA] * (2 * _NCHUNK)
    ),
)
def _identity_rows_sc(table_hbm, out_hbm, *scratch):
    slots = scratch[:_K]
    sem_in = scratch[_K:_K + _NCHUNK]
    sem_out = scratch[_K + _NCHUNK:]
    base = lax.axis_index("c") * _ROWS_PER_C

    ins = [None] * _NCHUNK
    outs = [None] * _NCHUNK

    def start_in(j):
        ins[j] = pltpu.async_copy(
            table_hbm.at[pl.ds(base + j * _S, _S)], slots[j % _K].at[:, 0],
            sem_in[j])

    def start_out(j):
        outs[j] = pltpu.async_copy(
            slots[j % _K], out_hbm.at[pl.ds(base + j * _S, _S)], sem_out[j])

    for j in range(_K):
        start_in(j)
    for j in range(_NCHUNK):
        ins[j].wait()
        start_out(j)
        if j + _K < _NCHUNK:
            outs[j].wait()
            start_in(j + _K)
    for j in range(_NCHUNK - _K, _NCHUNK):
        outs[j].wait()


def kernel(table):
    return _identity_rows_sc(table)
